# baseline (speedup 1.0000x reference)
import functools

import jax
import jax.numpy as jnp
from jax import lax
from jax.experimental import pallas as pl
from jax.experimental.pallas import tpu as pltpu
from jax.experimental.pallas import tpu_sc as plsc

_STRIDE = 18


def kernel(x):
    B, T, H, W, C = x.shape
    n_out = (T + _STRIDE - 1) // _STRIDE

    info = plsc.get_sparse_core_info()
    num_cores = info.num_cores

    mesh = plsc.VectorSubcoreMesh(core_axis_name="c", subcore_axis_name="s")

    @functools.partial(
        pl.kernel,
        mesh=mesh,
        out_type=jax.ShapeDtypeStruct((B, n_out, H, W, C), jnp.float32),
        compiler_params=pltpu.CompilerParams(use_tc_tiling_on_sc=True),
    )
    def copy_frames(x_hbm, out_hbm):
        wid = lax.axis_index("s") * num_cores + lax.axis_index("c")
        b = wid // n_out
        i = wid % n_out
        src = i * _STRIDE
        pltpu.sync_copy(x_hbm.at[b, src], out_hbm.at[b, i])

    return copy_frames(x)


# speedup vs baseline: 3.0042x; 3.0042x over previous
import functools

import jax
import jax.numpy as jnp
from jax import lax
from jax.experimental import pallas as pl
from jax.experimental.pallas import tpu as pltpu
from jax.experimental.pallas import tpu_sc as plsc

_STRIDE = 18


def kernel(x):
    B, T, H, W, C = x.shape
    n_out = (T + _STRIDE - 1) // _STRIDE

    info = plsc.get_sparse_core_info()
    num_cores = info.num_cores

    mesh = plsc.VectorSubcoreMesh(core_axis_name="c", subcore_axis_name="s")

    @functools.partial(
        pl.kernel,
        mesh=mesh,
        out_type=jax.ShapeDtypeStruct((B, n_out, H, W, C), jnp.float32),
        scratch_types=[pltpu.VMEM((H, W, C), jnp.float32)],
        compiler_params=pltpu.CompilerParams(use_tc_tiling_on_sc=True),
    )
    def copy_frames(x_hbm, out_hbm, buf):
        wid = lax.axis_index("s") * num_cores + lax.axis_index("c")
        b = wid // n_out
        i = wid % n_out
        src = i * _STRIDE
        pltpu.sync_copy(x_hbm.at[b, src], buf)
        pltpu.sync_copy(buf, out_hbm.at[b, i])

    return copy_frames(x)
